# Initial kernel scaffold; baseline (speedup 1.0000x reference)
import jax
import jax.numpy as jnp
from jax.experimental import pallas as pl


def kernel(node_fts, edge_fts, edge_index, F_norm_edge, F_dig, node_deg_vec, node_deg_mat, lap_mat, k_eig_val, k_eig_vec, num_nodes, norm_n, batch_idx, W_pre, b_pre, W_post, b_post):
    raise NotImplementedError("write your pallas kernel here")


# speedup vs baseline: 1.0543x; 1.0543x over previous
import functools

import jax
import jax.numpy as jnp
from jax import lax
from jax.experimental import pallas as pl
from jax.experimental.pallas import tpu as pltpu
from jax.experimental.pallas import tpu_sc as plsc

D = 128
NC = 2
NS = 16
NW = NC * NS
L = 16
C = 2000
G = 128
CAP = 2160


def _pre_body(x_ref, w_ref, b_ref, p_ref, q_ref):
    x = x_ref[...]
    w = w_ref[...]
    p_ref[...] = jnp.dot(x, w[:D, :], preferred_element_type=jnp.float32)
    q_ref[...] = jnp.dot(x, w[D:, :], preferred_element_type=jnp.float32) + b_ref[...]


def _post_body(x_ref, s_ref, mx_ref, dr_ref, deg_ref, fd_ref, nn_ref, w_ref,
               b_ref, o_ref):
    x = x_ref[...]
    deg = jnp.maximum(deg_ref[...], 1e-6)
    mean = s_ref[...] / deg
    dirv = dr_ref[...] - fd_ref[...] * mean
    w = w_ref[...]
    acc = jnp.dot(x, w[0 * D:1 * D, :], preferred_element_type=jnp.float32)
    acc = acc + jnp.dot(mean, w[1 * D:2 * D, :], preferred_element_type=jnp.float32)
    acc = acc + jnp.dot(mx_ref[...], w[2 * D:3 * D, :], preferred_element_type=jnp.float32)
    acc = acc + jnp.dot(dirv, w[3 * D:4 * D, :], preferred_element_type=jnp.float32)
    out = jnp.maximum(acc + b_ref[...], 0.0)
    o_ref[...] = x + nn_ref[...] * out


def _make_sc_edge_kernel(npt, nchunk):
    nps = npt * NS
    npad = npt * NW
    garb = nps

    def body(p_tbl, q_tbl, src_e, dst_e, f_e, zrows,
             sum_o, max_o, dir_o,
             acc_sum, acc_dir, maxacc,
             sdst, ssrc, sf, lsrc, ldl, lf,
             srcb, qidxb, idxb, mb, qb, dlb, fb, sem1, sem2):
        c = lax.axis_index("c")
        s = lax.axis_index("s")
        sc_base = c * nps
        gbase = sc_base + s * npt

        pltpu.sync_copy(zrows, maxacc)
        pltpu.sync_copy(zrows, acc_sum.at[pl.ds(s * npt, npt)])
        pltpu.sync_copy(zrows, acc_dir.at[pl.ds(s * npt, npt)])
        zero16i = jnp.zeros((L,), jnp.int32)

        def zinit(i, carry):
            lsrc[pl.ds(i * L, L)] = zero16i
            return carry

        lax.fori_loop(0, CAP // L, zinit, 0)

        def process(off, cnt):
            for k in range(G // L):
                sl = pl.ds(k * L, L)
                dlv = ldl[pl.ds(off + k * L, L)]
                idxb[sl] = dlv
                qidxb[sl] = dlv + sc_base
                srcb[sl] = lsrc[pl.ds(off + k * L, L)]
            cp1 = pltpu.async_copy(p_tbl.at[srcb], mb, sem1)
            cp2 = pltpu.async_copy(q_tbl.at[qidxb], qb, sem2)
            pltpu.sync_copy(ldl.at[pl.ds(off, G)], dlb)
            pltpu.sync_copy(lf.at[pl.ds(off, G)], fb)
            cp1.wait()
            cp2.wait()
            jreal = jnp.minimum(G, cnt - off)

            def jbody(j, carry):
                tl = dlb[j] - s * npt
                fj = fb[j]
                for k in range(D // L):
                    sl = pl.ds(k * L, L)
                    m = jnp.maximum(mb[j, sl] + qb[j, sl], 0.0)
                    mb[j, sl] = m
                    qb[j, sl] = fj * m
                    maxacc[tl, sl] = jnp.maximum(maxacc[tl, sl], m)
                return carry

            lax.fori_loop(0, jreal, jbody, 0)
            pltpu.sync_copy(mb, acc_sum.at[idxb], add=True)
            pltpu.sync_copy(qb, acc_dir.at[idxb], add=True)

        def chunk_body(ci, cnt):
            base = ci * C
            pltpu.sync_copy(dst_e.at[pl.ds(base, C)], sdst)
            pltpu.sync_copy(src_e.at[pl.ds(base, C)], ssrc)
            pltpu.sync_copy(f_e.at[pl.ds(base, C)], sf)

            def scan_body(i, cnt):
                sl = pl.ds(i * L, L)
                dv = sdst[sl]
                sv = ssrc[sl]
                fv = sf[sl]
                msk = (dv >= gbase) & (dv < gbase + npt)
                plsc.store_compressed(lsrc.at[pl.ds(cnt, L)], sv, mask=msk)
                plsc.store_compressed(ldl.at[pl.ds(cnt, L)], dv - sc_base, mask=msk)
                plsc.store_compressed(lf.at[pl.ds(cnt, L)], fv, mask=msk)
                return cnt + plsc.all_reduce_population_count(msk)[0]

            cnt = lax.fori_loop(0, C // L, scan_body, cnt)
            nb = cnt // G

            def bbody(b, carry):
                process(b * G, cnt)
                return carry

            lax.fori_loop(0, nb, bbody, 0)
            for k in range(G // L):
                sl = pl.ds(k * L, L)
                lsrc[sl] = lsrc[pl.ds(nb * G + k * L, L)]
                ldl[sl] = ldl[pl.ds(nb * G + k * L, L)]
                lf[sl] = lf[pl.ds(nb * G + k * L, L)]
            return cnt - nb * G

        cnt = lax.fori_loop(0, nchunk, chunk_body, jnp.int32(0))

        garbv = jnp.full((L,), garb, jnp.int32)
        for k in range(G // L):
            ldl[pl.ds(cnt + k * L, L)] = garbv
        process(0, cnt)

        pltpu.sync_copy(acc_sum.at[pl.ds(s * npt, npt)], sum_o.at[pl.ds(gbase, npt)])
        pltpu.sync_copy(acc_dir.at[pl.ds(s * npt, npt)], dir_o.at[pl.ds(gbase, npt)])
        pltpu.sync_copy(maxacc, max_o.at[pl.ds(gbase, npt)])

    return body, nps, npad


def kernel(node_fts, edge_fts, edge_index, F_norm_edge, F_dig, node_deg_vec,
           node_deg_mat, lap_mat, k_eig_val, k_eig_vec, num_nodes, norm_n,
           batch_idx, W_pre, b_pre, W_post, b_post):
    n = node_fts.shape[0]
    e = edge_index.shape[1]
    npt = -(-n // (NW * 8)) * 8
    nps = npt * NS
    npad = npt * NW
    rb = 1024
    tbl = -(-(2 * nps + 8) // rb) * rb

    x_pad = jnp.pad(node_fts, ((0, tbl - n), (0, 0)))
    b_pre2 = b_pre.reshape(1, D)

    p_tbl, q_tbl = pl.pallas_call(
        _pre_body,
        grid=(tbl // rb,),
        in_specs=[
            pl.BlockSpec((rb, D), lambda i: (i, 0)),
            pl.BlockSpec((2 * D, D), lambda i: (0, 0)),
            pl.BlockSpec((1, D), lambda i: (0, 0)),
        ],
        out_specs=[pl.BlockSpec((rb, D), lambda i: (i, 0))] * 2,
        out_shape=[jax.ShapeDtypeStruct((tbl, D), jnp.float32)] * 2,
    )(x_pad, W_pre, b_pre2)

    src = edge_index[0]
    dst = edge_index[1]
    m = jax.nn.relu(p_tbl[:n].at[src].get(mode="promise_in_bounds")
                    + q_tbl[:n].at[dst].get(mode="promise_in_bounds"))
    sum_f = jax.ops.segment_sum(m, dst, num_segments=n)
    mx = jax.ops.segment_max(m, dst, num_segments=n)
    mx = jnp.where(jnp.isfinite(mx), mx, 0.0)
    dir_f = jax.ops.segment_sum(F_norm_edge[:, None] * m, dst, num_segments=n)
    max_f = mx

    rb2 = 1000
    out = pl.pallas_call(
        _post_body,
        grid=(n // rb2,),
        in_specs=[
            pl.BlockSpec((rb2, D), lambda i: (i, 0)),
            pl.BlockSpec((rb2, D), lambda i: (i, 0)),
            pl.BlockSpec((rb2, D), lambda i: (i, 0)),
            pl.BlockSpec((rb2, D), lambda i: (i, 0)),
            pl.BlockSpec((rb2, 1), lambda i: (i, 0)),
            pl.BlockSpec((rb2, 1), lambda i: (i, 0)),
            pl.BlockSpec((rb2, 1), lambda i: (i, 0)),
            pl.BlockSpec((4 * D, D), lambda i: (0, 0)),
            pl.BlockSpec((1, D), lambda i: (0, 0)),
        ],
        out_specs=pl.BlockSpec((rb2, D), lambda i: (i, 0)),
        out_shape=jax.ShapeDtypeStruct((n, D), jnp.float32),
    )(node_fts, sum_f, max_f, dir_f, node_deg_vec,
      F_dig.reshape(n, 1), norm_n, W_post, b_post.reshape(1, D))
    return out


# speedup vs baseline: 1.5911x; 1.5091x over previous
import jax
import jax.numpy as jnp
from jax import lax
from jax.experimental import pallas as pl
from jax.experimental.pallas import tpu as pltpu
from jax.experimental.pallas import tpu_sc as plsc

D = 128
NC = 2
NS = 16
NW = NC * NS
L = 16
C = 2000
G = 128
CAP = 2192
TRASH = CAP - 1


def _pre_body(x_ref, w_ref, b_ref, p_ref, q_ref):
    x = x_ref[...]
    w = w_ref[...]
    p_ref[...] = jnp.dot(x, w[:D, :], preferred_element_type=jnp.float32)
    q_ref[...] = jnp.dot(x, w[D:, :], preferred_element_type=jnp.float32) + b_ref[...]


def _post_body(x_ref, s_ref, mx_ref, dr_ref, deg_ref, fd_ref, nn_ref, w_ref,
               b_ref, o_ref):
    x = x_ref[...]
    deg = jnp.maximum(deg_ref[...], 1e-6)
    mean = s_ref[...] / deg
    dirv = dr_ref[...] - fd_ref[...] * mean
    w = w_ref[...]
    acc = jnp.dot(x, w[0 * D:1 * D, :], preferred_element_type=jnp.float32)
    acc = acc + jnp.dot(mean, w[1 * D:2 * D, :], preferred_element_type=jnp.float32)
    acc = acc + jnp.dot(mx_ref[...], w[2 * D:3 * D, :], preferred_element_type=jnp.float32)
    acc = acc + jnp.dot(dirv, w[3 * D:4 * D, :], preferred_element_type=jnp.float32)
    out = jnp.maximum(acc + b_ref[...], 0.0)
    o_ref[...] = x + nn_ref[...] * out


def _prefix16(msk, iot):
    x = jnp.where(msk, 1, 0)
    for sft in (1, 2, 4, 8):
        g = x.at[jnp.maximum(iot - sft, 0)].get(mode="promise_in_bounds")
        x = jnp.where(iot >= sft, x + g, x)
    tot = x.at[jnp.full((L,), L - 1, jnp.int32)].get(mode="promise_in_bounds")
    return x, tot


def _splat(v, j):
    return v.at[jnp.full((L,), j, jnp.int32)].get(mode="promise_in_bounds")


def _make_sc_sumdir_kernel(npt, nchunk):
    nps = npt * NS
    garb = nps

    def body(p_tbl, q_tbl, src_e, dst_e, f_e, zrows,
             sum_o, dir_o,
             acc_sum, acc_dir,
             sdst, ssrc, sf, lsrc, ldl, lf,
             srcb, qidxb, idxb, tmpv, mb, qb, sem1, sem2, semd):
        c = lax.axis_index("c")
        s = lax.axis_index("s")
        sc_base = c * nps
        gbase = sc_base + s * npt
        iot = lax.iota(jnp.int32, L)

        pltpu.sync_copy(zrows, acc_sum.at[pl.ds(s * npt, npt)])
        pltpu.sync_copy(zrows, acc_dir.at[pl.ds(s * npt, npt)])
        zero16i = jnp.zeros((L,), jnp.int32)

        def zinit(i, carry):
            lsrc[pl.ds(i * L, L)] = zero16i
            ldl[pl.ds(i * L, L)] = zero16i
            return carry

        lax.fori_loop(0, CAP // L, zinit, 0)

        def process(off, cnt):
            for k in range(G // L):
                sl = pl.ds(k * L, L)
                dlv = ldl[pl.ds(off + k * L, L)]
                idxb[sl] = dlv
                qidxb[sl] = dlv + sc_base
                srcb[sl] = lsrc[pl.ds(off + k * L, L)]
            cp1 = pltpu.async_copy(p_tbl.at[srcb], mb, sem1)
            cp2 = pltpu.async_copy(q_tbl.at[qidxb], qb, sem2)
            cp1.wait()
            cp2.wait()
            jreal = jnp.minimum(G, cnt - off)
            for g16 in range(G // L):
                fv = lf[pl.ds(off + g16 * L, L)]

                def jbody(jj, carry):
                    fj = _splat(fv, jj)
                    jx = g16 * L + jj
                    for k in range(D // L):
                        sl = pl.ds(k * L, L)
                        m = jnp.maximum(mb[jx, sl] + qb[jx, sl], 0.0)
                        mb[jx, sl] = m
                        qb[jx, sl] = fj * m
                    return carry

                jlim = jnp.clip(jreal - g16 * L, 0, L)
                lax.fori_loop(0, jlim, jbody, 0)
            pltpu.sync_copy(mb, acc_sum.at[idxb], add=True)
            pltpu.sync_copy(qb, acc_dir.at[idxb], add=True)

        def chunk_body(ci, cnt_spl):
            base = ci * C
            cpd = pltpu.async_copy(dst_e.at[pl.ds(base, C)], sdst, semd)
            cps = pltpu.async_copy(src_e.at[pl.ds(base, C)], ssrc, sem1)
            cpf = pltpu.async_copy(f_e.at[pl.ds(base, C)], sf, sem2)
            cpd.wait()
            cps.wait()
            cpf.wait()

            def scan_body(i, cnt_spl):
                sl = pl.ds(i * L, L)
                dv = sdst[sl]
                msk = (dv >= gbase) & (dv < gbase + npt)
                x, tot = _prefix16(msk, iot)
                pos = jnp.where(msk, cnt_spl + x - 1, TRASH)
                plsc.store_scatter(lsrc, [pos], ssrc[sl])
                plsc.store_scatter(ldl, [pos], dv - sc_base)
                plsc.store_scatter(lf, [pos], sf[sl])
                return cnt_spl + tot

            cnt_spl = lax.fori_loop(0, C // L, scan_body, cnt_spl)
            tmpv[pl.ds(0, L)] = cnt_spl
            cnt = tmpv[pl.ds(0, L)][0]
            nb = cnt // G

            def bbody(b, carry):
                process(b * G, cnt)
                return carry

            lax.fori_loop(0, nb, bbody, 0)
            for k in range(G // L):
                sl = pl.ds(k * L, L)
                lsrc[sl] = lsrc[pl.ds(nb * G + k * L, L)]
                ldl[sl] = ldl[pl.ds(nb * G + k * L, L)]
                lf[sl] = lf[pl.ds(nb * G + k * L, L)]
            return cnt_spl - nb * G

        cnt_spl = lax.fori_loop(0, nchunk, chunk_body, jnp.zeros((L,), jnp.int32))
        tmpv[pl.ds(0, L)] = cnt_spl
        cnt = tmpv[pl.ds(0, L)][0]

        garbv = jnp.full((L,), garb, jnp.int32)
        for k in range(G // L):
            ldl[pl.ds(cnt + k * L, L)] = garbv
        process(0, cnt)

        pltpu.sync_copy(acc_sum.at[pl.ds(s * npt, npt)], sum_o.at[pl.ds(gbase, npt)])
        pltpu.sync_copy(acc_dir.at[pl.ds(s * npt, npt)], dir_o.at[pl.ds(gbase, npt)])

    return body


def _make_sc_max_kernel(npt, nchunk):
    nps = npt * NS

    def body(p_tbl, q_tbl, src_e, dst_e, zrows,
             max_o,
             maxacc,
             sdst, ssrc, lsrc, ldl,
             srcb, qidxb, tmpv, mb, qb, sem1, sem2, semd):
        c = lax.axis_index("c")
        s = lax.axis_index("s")
        sc_base = c * nps
        gbase = sc_base + s * npt
        iot = lax.iota(jnp.int32, L)

        pltpu.sync_copy(zrows, maxacc)
        zero16i = jnp.zeros((L,), jnp.int32)

        def zinit(i, carry):
            lsrc[pl.ds(i * L, L)] = zero16i
            ldl[pl.ds(i * L, L)] = zero16i
            return carry

        lax.fori_loop(0, CAP // L, zinit, 0)

        def process(off, cnt):
            for k in range(G // L):
                sl = pl.ds(k * L, L)
                qidxb[sl] = ldl[pl.ds(off + k * L, L)] + sc_base
                srcb[sl] = lsrc[pl.ds(off + k * L, L)]
            cp1 = pltpu.async_copy(p_tbl.at[srcb], mb, sem1)
            cp2 = pltpu.async_copy(q_tbl.at[qidxb], qb, sem2)
            cp1.wait()
            cp2.wait()
            jreal = jnp.minimum(G, cnt - off)
            for g16 in range(G // L):
                dlv = ldl[pl.ds(off + g16 * L, L)]

                def jbody(jj, carry):
                    tl = _splat(dlv, jj)[0] - s * npt
                    jx = g16 * L + jj
                    for k in range(D // L):
                        sl = pl.ds(k * L, L)
                        m = jnp.maximum(mb[jx, sl] + qb[jx, sl], 0.0)
                        maxacc[tl, sl] = jnp.maximum(maxacc[tl, sl], m)
                    return carry

                jlim = jnp.clip(jreal - g16 * L, 0, L)
                lax.fori_loop(0, jlim, jbody, 0)

        def chunk_body(ci, cnt_spl):
            base = ci * C
            cpd = pltpu.async_copy(dst_e.at[pl.ds(base, C)], sdst, semd)
            cps = pltpu.async_copy(src_e.at[pl.ds(base, C)], ssrc, sem1)
            cpd.wait()
            cps.wait()

            def scan_body(i, cnt_spl):
                sl = pl.ds(i * L, L)
                dv = sdst[sl]
                msk = (dv >= gbase) & (dv < gbase + npt)
                x, tot = _prefix16(msk, iot)
                pos = jnp.where(msk, cnt_spl + x - 1, TRASH)
                plsc.store_scatter(lsrc, [pos], ssrc[sl])
                plsc.store_scatter(ldl, [pos], dv - sc_base)
                return cnt_spl + tot

            cnt_spl = lax.fori_loop(0, C // L, scan_body, cnt_spl)
            tmpv[pl.ds(0, L)] = cnt_spl
            cnt = tmpv[pl.ds(0, L)][0]
            nb = cnt // G

            def bbody(b, carry):
                process(b * G, cnt)
                return carry

            lax.fori_loop(0, nb, bbody, 0)
            for k in range(G // L):
                sl = pl.ds(k * L, L)
                lsrc[sl] = lsrc[pl.ds(nb * G + k * L, L)]
                ldl[sl] = ldl[pl.ds(nb * G + k * L, L)]
            return cnt_spl - nb * G

        cnt_spl = lax.fori_loop(0, nchunk, chunk_body, jnp.zeros((L,), jnp.int32))
        tmpv[pl.ds(0, L)] = cnt_spl
        cnt = tmpv[pl.ds(0, L)][0]
        process(0, cnt)

        pltpu.sync_copy(maxacc, max_o.at[pl.ds(gbase, npt)])

    return body


def kernel(node_fts, edge_fts, edge_index, F_norm_edge, F_dig, node_deg_vec,
           node_deg_mat, lap_mat, k_eig_val, k_eig_vec, num_nodes, norm_n,
           batch_idx, W_pre, b_pre, W_post, b_post):
    n = node_fts.shape[0]
    e = edge_index.shape[1]
    npt = -(-n // (NW * 8)) * 8
    nps = npt * NS
    npad = npt * NW
    rb = 1024
    tbl = -(-(2 * nps + 8) // rb) * rb

    x_pad = jnp.pad(node_fts, ((0, tbl - n), (0, 0)))
    b_pre2 = b_pre.reshape(1, D)

    p_tbl, q_tbl = pl.pallas_call(
        _pre_body,
        grid=(tbl // rb,),
        in_specs=[
            pl.BlockSpec((rb, D), lambda i: (i, 0)),
            pl.BlockSpec((2 * D, D), lambda i: (0, 0)),
            pl.BlockSpec((1, D), lambda i: (0, 0)),
        ],
        out_specs=[pl.BlockSpec((rb, D), lambda i: (i, 0))] * 2,
        out_shape=[jax.ShapeDtypeStruct((tbl, D), jnp.float32)] * 2,
    )(x_pad, W_pre, b_pre2)

    epad = -(-e // C) * C
    src_e = jnp.pad(edge_index[0], (0, epad - e))
    dst_e = jnp.pad(edge_index[1], (0, epad - e), constant_values=npad - 1)
    f_e = jnp.pad(F_norm_edge, (0, epad - e))
    nchunk = epad // C

    mesh = plsc.VectorSubcoreMesh(core_axis_name="c", subcore_axis_name="s")
    zrows = jnp.zeros((npt, D), jnp.float32)
    cparams = pltpu.CompilerParams(needs_layout_passes=False)

    sum_f, dir_f = pl.kernel(
        _make_sc_sumdir_kernel(npt, nchunk),
        out_type=[jax.ShapeDtypeStruct((npad, D), jnp.float32)] * 2,
        mesh=mesh,
        compiler_params=cparams,
        scratch_types=[
            pltpu.VMEM_SHARED((nps + 8, D), jnp.float32),
            pltpu.VMEM_SHARED((nps + 8, D), jnp.float32),
            pltpu.VMEM((C,), jnp.int32),
            pltpu.VMEM((C,), jnp.int32),
            pltpu.VMEM((C,), jnp.float32),
            pltpu.VMEM((CAP,), jnp.int32),
            pltpu.VMEM((CAP,), jnp.int32),
            pltpu.VMEM((CAP,), jnp.float32),
            pltpu.VMEM((G,), jnp.int32),
            pltpu.VMEM((G,), jnp.int32),
            pltpu.VMEM((G,), jnp.int32),
            pltpu.VMEM((L,), jnp.int32),
            pltpu.VMEM((G, D), jnp.float32),
            pltpu.VMEM((G, D), jnp.float32),
            pltpu.SemaphoreType.DMA,
            pltpu.SemaphoreType.DMA,
            pltpu.SemaphoreType.DMA,
        ],
    )(p_tbl, q_tbl, src_e, dst_e, f_e, zrows)

    max_f = pl.kernel(
        _make_sc_max_kernel(npt, nchunk),
        out_type=jax.ShapeDtypeStruct((npad, D), jnp.float32),
        mesh=mesh,
        compiler_params=cparams,
        scratch_types=[
            pltpu.VMEM((npt, D), jnp.float32),
            pltpu.VMEM((C,), jnp.int32),
            pltpu.VMEM((C,), jnp.int32),
            pltpu.VMEM((CAP,), jnp.int32),
            pltpu.VMEM((CAP,), jnp.int32),
            pltpu.VMEM((G,), jnp.int32),
            pltpu.VMEM((G,), jnp.int32),
            pltpu.VMEM((L,), jnp.int32),
            pltpu.VMEM((G, D), jnp.float32),
            pltpu.VMEM((G, D), jnp.float32),
            pltpu.SemaphoreType.DMA,
            pltpu.SemaphoreType.DMA,
            pltpu.SemaphoreType.DMA,
        ],
    )(p_tbl, q_tbl, src_e, dst_e, zrows)

    rb2 = 1000
    out = pl.pallas_call(
        _post_body,
        grid=(n // rb2,),
        in_specs=[
            pl.BlockSpec((rb2, D), lambda i: (i, 0)),
            pl.BlockSpec((rb2, D), lambda i: (i, 0)),
            pl.BlockSpec((rb2, D), lambda i: (i, 0)),
            pl.BlockSpec((rb2, D), lambda i: (i, 0)),
            pl.BlockSpec((rb2, 1), lambda i: (i, 0)),
            pl.BlockSpec((rb2, 1), lambda i: (i, 0)),
            pl.BlockSpec((rb2, 1), lambda i: (i, 0)),
            pl.BlockSpec((4 * D, D), lambda i: (0, 0)),
            pl.BlockSpec((1, D), lambda i: (0, 0)),
        ],
        out_specs=pl.BlockSpec((rb2, D), lambda i: (i, 0)),
        out_shape=jax.ShapeDtypeStruct((n, D), jnp.float32),
    )(node_fts, sum_f[:n], max_f[:n], dir_f[:n], node_deg_vec,
      F_dig.reshape(n, 1), norm_n, W_post, b_post.reshape(1, D))
    return out
